# Initial kernel scaffold; baseline (speedup 1.0000x reference)
import jax
import jax.numpy as jnp
from jax.experimental import pallas as pl


def kernel(nodes, to_neighs, batch_trans_features, feat_table, fc_W, fc_b, weight, relations_atten):
    raise NotImplementedError("write your pallas kernel here")


# speedup vs baseline: 3.8820x; 3.8820x over previous
import functools

import jax
import jax.numpy as jnp
from jax.experimental import pallas as pl

N = 4096
B = 1024
K = 16
FEAT = 256
EMB = 128
R = 3
M = K + 3
MP = 24
BB = 256

_INTERPRET = False


def _prep_body(nodes_ref, neighs_ref, btf_ref, pm_ref, invde_ref):
    nodes_col = nodes_ref[...]
    btf = btf_ref[...]
    lane = jax.lax.broadcasted_iota(jnp.int32, (BB, N), 1)
    onehot = (nodes_col == lane).astype(jnp.float32)
    q = jnp.dot(onehot, btf, preferred_element_type=jnp.float32)
    sim = jax.lax.dot_general(q, btf, (((1,), (1,)), ((), ())),
                              preferred_element_type=jnp.float32) - onehot
    m1 = jnp.max(sim, axis=1, keepdims=True)
    i1 = jnp.min(jnp.where(sim == m1, lane, N), axis=1, keepdims=True)
    sim_m = jnp.where(lane == i1, -3e38, sim)
    m2 = jnp.max(sim_m, axis=1, keepdims=True)
    i2 = jnp.min(jnp.where(sim_m == m2, lane, N), axis=1, keepdims=True)

    for r in range(R):
        mem = jnp.concatenate(
            [neighs_ref[r], i1, i2, nodes_col], axis=1)
        cols = [mem[:, j:j + 1] for j in range(M)]
        pm_cols = [cols[0]]
        de = jnp.ones((BB, 1), jnp.float32)
        for j in range(1, M):
            dup = cols[0] == cols[j]
            for i in range(1, j):
                dup = dup | (cols[i] == cols[j])
            pm_cols.append(jnp.where(dup, N, cols[j]))
            de = de + jnp.where(dup, 0.0, 1.0)
        pm = jnp.concatenate(
            pm_cols + [jnp.full((BB, MP - M), N, jnp.int32)], axis=1)
        pm_ref[r] = pm
        invde_ref[r] = 1.0 / de


def _chain_body(pm_ref, nodes_ref, feat_ref, fcw_ref, fcb_ref, w_ref,
                invde_ref, out_ref):
    r = pl.program_id(0)
    pm = pm_ref[0]
    nodes_col = nodes_ref[...]
    invde = invde_ref[0]
    lane = jax.lax.broadcasted_iota(jnp.int32, (B, N), 1)

    ht_b = pm[:, 0:1] == lane
    for j in range(1, M):
        ht_b = ht_b | (pm[:, j:j + 1] == lane)
    ht = ht_b.astype(jnp.float32)
    dv = jnp.sum(ht, axis=0, keepdims=True)
    pos = dv > 0
    dv2 = jnp.where(pos, jax.lax.rsqrt(dv), 0.0)
    invdv = jnp.where(pos, 1.0 / dv, 0.0)
    hs = ht * dv2

    x = jnp.dot(feat_ref[...], fcw_ref[0], preferred_element_type=jnp.float32)
    x = x + fcb_ref[0]
    x = jnp.where(x >= 0, x, 0.01 * x)

    t1 = invde * jnp.dot(hs, x, preferred_element_type=jnp.float32)
    yy = jax.lax.dot_general(hs, t1, (((0,), (0,)), ((), ())),
                             preferred_element_type=jnp.float32)
    t2 = invde * jnp.dot(ht, yy, preferred_element_type=jnp.float32)
    zz = jax.lax.dot_general(ht, t2, (((0,), (0,)), ((), ())),
                             preferred_element_type=jnp.float32)
    oi = (nodes_col == lane).astype(jnp.float32) * invdv
    vxq = jnp.dot(oi, zz, preferred_element_type=jnp.float32)
    contrib = jax.nn.relu(
        jnp.dot(vxq, w_ref[0], preferred_element_type=jnp.float32))

    @pl.when(r == 0)
    def _():
        out_ref[...] = jnp.zeros_like(out_ref)

    out_ref[...] += contrib


def kernel(nodes, to_neighs, batch_trans_features, feat_table, fc_W, fc_b,
           weight, relations_atten):
    del relations_atten
    nodes_col = nodes.astype(jnp.int32).reshape(B, 1)
    neighs = to_neighs.astype(jnp.int32)

    pm, invde = pl.pallas_call(
        _prep_body,
        grid=(B // BB,),
        in_specs=[
            pl.BlockSpec((BB, 1), lambda i: (i, 0)),
            pl.BlockSpec((R, BB, K), lambda i: (0, i, 0)),
            pl.BlockSpec((N, EMB), lambda i: (0, 0)),
        ],
        out_specs=[
            pl.BlockSpec((R, BB, MP), lambda i: (0, i, 0)),
            pl.BlockSpec((R, BB, 1), lambda i: (0, i, 0)),
        ],
        out_shape=[
            jax.ShapeDtypeStruct((R, B, MP), jnp.int32),
            jax.ShapeDtypeStruct((R, B, 1), jnp.float32),
        ],
        interpret=_INTERPRET,
    )(nodes_col, neighs, batch_trans_features)

    out = pl.pallas_call(
        _chain_body,
        grid=(R,),
        in_specs=[
            pl.BlockSpec((1, B, MP), lambda r: (r, 0, 0)),
            pl.BlockSpec((B, 1), lambda r: (0, 0)),
            pl.BlockSpec((N, FEAT), lambda r: (0, 0)),
            pl.BlockSpec((1, FEAT, EMB), lambda r: (r, 0, 0)),
            pl.BlockSpec((1, 1, EMB), lambda r: (r, 0, 0)),
            pl.BlockSpec((1, EMB, EMB), lambda r: (r, 0, 0)),
            pl.BlockSpec((1, B, 1), lambda r: (r, 0, 0)),
        ],
        out_specs=pl.BlockSpec((B, EMB), lambda r: (0, 0)),
        out_shape=jax.ShapeDtypeStruct((B, EMB), jnp.float32),
        interpret=_INTERPRET,
    )(pm, nodes_col, feat_table, fc_W, fc_b.reshape(R, 1, EMB), weight,
      invde)
    return out
